# baseline (speedup 1.0000x reference)
import functools

import jax
import jax.numpy as jnp
from jax.experimental import pallas as pl
from jax.experimental.pallas import tpu as pltpu

_NUM_EXPERTS = 16
_TOP_K = 2
_LOAD_BALANCE_COEF = 0.01
_Z_LOSS_COEF = 0.001
_EPS = 1e-6


def _router_body(x_ref, w_ref, logits_ref, ew_ref, ei_ref, aux_ref,
                 cnt_acc, sp_acc, z_acc, *, num_t, num_h, total_tokens):
    ti = pl.program_id(0)
    hi = pl.program_id(1)

    x = x_ref[...]
    w = w_ref[...]
    part = jax.lax.dot_general(
        w, x, (((1,), (1,)), ((), ())),
        preferred_element_type=jnp.float32)

    @pl.when(hi == 0)
    def _first():
        logits_ref[...] = part

    @pl.when(hi > 0)
    def _rest():
        logits_ref[...] += part

    @pl.when(hi == num_h - 1)
    def _epilogue():
        lt = logits_ref[...]

        m = jnp.max(lt, axis=0, keepdims=True)
        e = jnp.exp(lt - m)
        s = jnp.sum(e, axis=0, keepdims=True)

        iota = jax.lax.broadcasted_iota(jnp.int32, lt.shape, 0)
        i1 = jnp.min(jnp.where(lt == m, iota, _NUM_EXPERTS),
                     axis=0, keepdims=True)
        masked = jnp.where(iota == i1, -jnp.inf, lt)
        v2 = jnp.max(masked, axis=0, keepdims=True)
        i2 = jnp.min(jnp.where(masked == v2, iota, _NUM_EXPERTS),
                     axis=0, keepdims=True)

        rs = 1.0 / s
        p1 = rs
        p2 = jnp.exp(v2 - m) * rs
        rden = 1.0 / (p1 + p2 + _EPS)
        ew_ref[...] = jnp.concatenate([p1 * rden, p2 * rden], axis=0)
        ei_ref[...] = jnp.concatenate([i1, i2], axis=0)

        one_hot = ((iota == i1) | (iota == i2)).astype(jnp.float32)
        cnt_tile = jnp.sum(one_hot, axis=1, keepdims=True)
        sp_tile = jnp.sum(e * rs, axis=1, keepdims=True)
        lse = m + jnp.log(s)
        z_tile = jnp.sum(lse * lse, axis=1, keepdims=True)

        @pl.when(ti == 0)
        def _init():
            cnt_acc[...] = cnt_tile
            sp_acc[...] = sp_tile
            z_acc[...] = z_tile

        @pl.when(ti > 0)
        def _accum():
            cnt_acc[...] += cnt_tile
            sp_acc[...] += sp_tile
            z_acc[...] += z_tile

        @pl.when(ti == num_t - 1)
        def _finalize():
            t = jnp.float32(total_tokens)
            lb = jnp.sum(cnt_acc[...] * sp_acc[...], axis=0, keepdims=True)
            lb = lb * (_NUM_EXPERTS / (t * t))
            aux_ref[...] = (_LOAD_BALANCE_COEF * lb
                            + (_Z_LOSS_COEF / t) * z_acc[...])


@jax.jit
def kernel(hidden_states, W):
    B, S, H = hidden_states.shape
    T = B * S
    E = _NUM_EXPERTS
    x = hidden_states.reshape(T, H)

    block_t = 1024
    block_h = 1024
    num_t = T // block_t
    num_h = H // block_h

    logits, ew, ei, aux = pl.pallas_call(
        functools.partial(_router_body, num_t=num_t, num_h=num_h,
                          total_tokens=T),
        grid=(num_t, num_h),
        in_specs=[
            pl.BlockSpec((block_t, block_h), lambda i, h: (i, h)),
            pl.BlockSpec((E, block_h), lambda i, h: (0, h)),
        ],
        out_specs=[
            pl.BlockSpec((E, block_t), lambda i, h: (0, i)),
            pl.BlockSpec((_TOP_K, block_t), lambda i, h: (0, i)),
            pl.BlockSpec((_TOP_K, block_t), lambda i, h: (0, i)),
            pl.BlockSpec((1, 1), lambda i, h: (0, 0)),
        ],
        out_shape=[
            jax.ShapeDtypeStruct((E, T), jnp.float32),
            jax.ShapeDtypeStruct((_TOP_K, T), jnp.float32),
            jax.ShapeDtypeStruct((_TOP_K, T), jnp.int32),
            jax.ShapeDtypeStruct((1, 1), jnp.float32),
        ],
        scratch_shapes=[
            pltpu.VMEM((E, 1), jnp.float32),
            pltpu.VMEM((E, 1), jnp.float32),
            pltpu.VMEM((1, 1), jnp.float32),
        ],
    )(x, W)

    return logits.T, ew.T, ei.T, aux[0, 0]


# speedup vs baseline: 1.2129x; 1.2129x over previous
import functools

import jax
import jax.numpy as jnp
from jax.experimental import pallas as pl
from jax.experimental.pallas import tpu as pltpu

_NUM_EXPERTS = 16
_TOP_K = 2
_LOAD_BALANCE_COEF = 0.01
_Z_LOSS_COEF = 0.001
_EPS = 1e-6


def _router_body(x_ref, w_ref, logits_ref, ew_ref, ei_ref, aux_ref,
                 cnt_acc, sp_acc, z_acc, *, num_steps, total_tokens):
    pi = pl.program_id(0)

    x = x_ref[...]
    w = w_ref[...]
    lt = jax.lax.dot_general(
        w, x, (((1,), (1,)), ((), ())),
        preferred_element_type=jnp.float32)
    logits_ref[...] = lt

    m = jnp.max(lt, axis=0, keepdims=True)
    e = jnp.exp(lt - m)
    s = jnp.sum(e, axis=0, keepdims=True)

    iota = jax.lax.broadcasted_iota(jnp.int32, lt.shape, 0)
    i1 = jnp.min(jnp.where(lt == m, iota, _NUM_EXPERTS),
                 axis=0, keepdims=True)
    masked = jnp.where(iota == i1, -jnp.inf, lt)
    v2 = jnp.max(masked, axis=0, keepdims=True)
    i2 = jnp.min(jnp.where(masked == v2, iota, _NUM_EXPERTS),
                 axis=0, keepdims=True)

    rs = 1.0 / s
    p1 = rs
    p2 = jnp.exp(v2 - m) * rs
    rden = 1.0 / (p1 + p2 + _EPS)
    ew_ref[...] = jnp.concatenate([p1 * rden, p2 * rden], axis=0)
    ei_ref[...] = jnp.concatenate([i1, i2], axis=0)

    one_hot = ((iota == i1) | (iota == i2)).astype(jnp.float32)
    cnt_tile = jnp.sum(one_hot, axis=1, keepdims=True)
    sp_tile = jnp.sum(e * rs, axis=1, keepdims=True)
    lse = m + jnp.log(s)
    z_tile = jnp.sum(lse * lse, axis=1, keepdims=True)

    @pl.when(pi == 0)
    def _init():
        cnt_acc[...] = cnt_tile
        sp_acc[...] = sp_tile
        z_acc[...] = z_tile

    @pl.when(pi > 0)
    def _accum():
        cnt_acc[...] += cnt_tile
        sp_acc[...] += sp_tile
        z_acc[...] += z_tile

    @pl.when(pi == num_steps - 1)
    def _finalize():
        t = jnp.float32(total_tokens)
        lb = jnp.sum(cnt_acc[...] * sp_acc[...], axis=0, keepdims=True)
        lb = lb * (_NUM_EXPERTS / (t * t))
        aux_ref[...] = _LOAD_BALANCE_COEF * lb + (_Z_LOSS_COEF / t) * z_acc[...]


@jax.jit
def kernel(hidden_states, W):
    B, S, H = hidden_states.shape
    T = B * S
    E = _NUM_EXPERTS
    x = hidden_states.reshape(T, H)

    block_t = 1024
    num_steps = T // block_t

    logits, ew, ei, aux = pl.pallas_call(
        functools.partial(_router_body, num_steps=num_steps, total_tokens=T),
        grid=(num_steps,),
        in_specs=[
            pl.BlockSpec((block_t, H), lambda i: (i, 0)),
            pl.BlockSpec((E, H), lambda i: (0, 0)),
        ],
        out_specs=[
            pl.BlockSpec((E, block_t), lambda i: (0, i)),
            pl.BlockSpec((_TOP_K, block_t), lambda i: (0, i)),
            pl.BlockSpec((_TOP_K, block_t), lambda i: (0, i)),
            pl.BlockSpec((1, 1), lambda i: (0, 0)),
        ],
        out_shape=[
            jax.ShapeDtypeStruct((E, T), jnp.float32),
            jax.ShapeDtypeStruct((_TOP_K, T), jnp.float32),
            jax.ShapeDtypeStruct((_TOP_K, T), jnp.int32),
            jax.ShapeDtypeStruct((1, 1), jnp.float32),
        ],
        scratch_shapes=[
            pltpu.VMEM((E, 1), jnp.float32),
            pltpu.VMEM((E, 1), jnp.float32),
            pltpu.VMEM((1, 1), jnp.float32),
        ],
    )(x, W)

    return logits.T, ew.T, ei.T, aux[0, 0]
